# baseline (speedup 1.0000x reference)
import functools

import jax
import jax.numpy as jnp
import numpy as np
from jax import lax
from jax.experimental import pallas as pl
from jax.experimental.pallas import tpu as pltpu
from jax.experimental.pallas import tpu_sc as plsc

B = 4096
SEQ = 200
DIM = 64

NC = 2
NS = 16
NW = NC * NS
B_PER_W = B // NW
CH0 = 104
CH1 = 96
RING = 4

_PERM = np.concatenate([np.arange(0, 32, 2), np.arange(1, 32, 2),
                        np.arange(32, 64, 2), np.arange(33, 64, 2)])


def _sc_pool_call(x2, table_bf):
    mesh = plsc.VectorSubcoreMesh(
        core_axis_name="c", subcore_axis_name="s", num_cores=NC,
        num_subcores=NS)

    @functools.partial(
        pl.kernel,
        out_type=jax.ShapeDtypeStruct((B, DIM), jnp.float32),
        mesh=mesh,
        compiler_params=pltpu.CompilerParams(use_tc_tiling_on_sc=False,
                                             needs_layout_passes=False),
        scratch_types=[
            pltpu.VMEM((B_PER_W, SEQ), jnp.int32),
            *[pltpu.VMEM(((CH0, CH1)[b % 2], DIM), jnp.bfloat16)
              for b in range(RING)],
            pltpu.VMEM((B_PER_W, DIM), jnp.float32),
            *[pltpu.SemaphoreType.DMA for _ in range(RING)],
        ],
    )
    def sc_pool(x_hbm, table_hbm, out_hbm, idx_v, r0, r1, r2, r3,
                pool_v, s0, s1, s2, s3):
        rows = (r0, r1, r2, r3)
        sems = (s0, s1, s2, s3)
        chlen = (CH0, CH1)
        choff = (0, CH0)
        wid = lax.axis_index("s") * NC + lax.axis_index("c")
        base = wid * B_PER_W
        pltpu.sync_copy(x_hbm.at[pl.ds(base, B_PER_W)], idx_v)

        for b in range(RING):
            pltpu.async_copy(
                table_hbm.at[idx_v.at[b // 2, pl.ds(choff[b % 2],
                                                    chlen[b % 2])]],
                rows[b], sems[b])

        scale = jnp.float32(1.0 / SEQ)
        zero = jnp.zeros((16,), jnp.float32)
        samples_per_group = RING // 2

        def outer(t, _):
            for k in range(samples_per_group):
                s = samples_per_group * t + k
                accs = (zero, zero, zero, zero)
                for hb in range(2):
                    b = 2 * k + hb
                    pltpu.make_async_copy(
                        table_hbm.at[pl.ds(0, chlen[hb])],
                        rows[b], sems[b]).wait()

                    rbuf = rows[b]

                    def acc_body(r, a, rbuf=rbuf):
                        c0 = rbuf[r, pl.ds(0, 32)]
                        c1 = rbuf[r, pl.ds(32, 32)]
                        e0, o0 = plsc.unpack(
                            c0, format=plsc.PackFormat.INTERLEAVED)
                        e1, o1 = plsc.unpack(
                            c1, format=plsc.PackFormat.INTERLEAVED)
                        return (a[0] + e0, a[1] + o0, a[2] + e1, a[3] + o1)

                    accs = lax.fori_loop(0, chlen[hb], acc_body, accs,
                                         unroll=8)

                    s_next = s + samples_per_group

                    @pl.when(s_next < B_PER_W)
                    def _(b=b, hb=hb, s_next=s_next):
                        pltpu.async_copy(
                            table_hbm.at[idx_v.at[s_next,
                                                  pl.ds(choff[hb],
                                                        chlen[hb])]],
                            rows[b], sems[b])

                pool_v[s, pl.ds(0, 16)] = accs[0] * scale
                pool_v[s, pl.ds(16, 16)] = accs[1] * scale
                pool_v[s, pl.ds(32, 16)] = accs[2] * scale
                pool_v[s, pl.ds(48, 16)] = accs[3] * scale
            return 0

        lax.fori_loop(0, B_PER_W // samples_per_group, outer, 0)
        pltpu.sync_copy(pool_v, out_hbm.at[pl.ds(base, B_PER_W)])

    return sc_pool(x2, table_bf)


def _mlp_body(m_ref, w1_ref, b1_ref, w2_ref, b2_ref, o_ref):
    h = jnp.dot(m_ref[...], w1_ref[...],
                preferred_element_type=jnp.float32) + b1_ref[...]
    h = jax.nn.sigmoid(jnp.maximum(h, 0.0))
    o = jnp.dot(h, w2_ref[...],
                preferred_element_type=jnp.float32) + b2_ref[...]
    o_ref[...] = jax.nn.sigmoid(o)


def _mlp_call(pooled, W1, b1, W2, b2):
    return pl.pallas_call(
        _mlp_body,
        out_shape=jax.ShapeDtypeStruct((B, 1), jnp.float32),
    )(pooled, W1, b1.reshape(1, 16), W2, b2.reshape(1, 1))


def kernel(x, table, W1, b1, W2, b2):
    pooled = _sc_pool_call(x.astype(jnp.int32), table.astype(jnp.bfloat16))
    W1p = W1[jnp.asarray(_PERM), :]
    return _mlp_call(pooled, W1p, b1, W2, b2)


# speedup vs baseline: 1.6087x; 1.6087x over previous
import functools

import jax
import jax.numpy as jnp
from jax import lax
from jax.experimental import pallas as pl
from jax.experimental.pallas import tpu as pltpu
from jax.experimental.pallas import tpu_sc as plsc

B = 4096
SEQ = 200
DIM = 64

NC = 2
NS = 16
NW = NC * NS
B_PER_W = B // NW
CH0 = 104
CH1 = 96
RING = 8


def _sc_pool_call(x2, table):
    mesh = plsc.VectorSubcoreMesh(
        core_axis_name="c", subcore_axis_name="s", num_cores=NC,
        num_subcores=NS)

    @functools.partial(
        pl.kernel,
        out_type=jax.ShapeDtypeStruct((B, DIM), jnp.float32),
        mesh=mesh,
        compiler_params=pltpu.CompilerParams(use_tc_tiling_on_sc=False),
        scratch_types=[
            pltpu.VMEM((B_PER_W, SEQ), jnp.int32),
            *[pltpu.VMEM(((CH0, CH1)[b % 2], DIM), jnp.float32)
              for b in range(RING)],
            pltpu.VMEM((B_PER_W, DIM), jnp.float32),
            *[pltpu.SemaphoreType.DMA for _ in range(RING)],
        ],
    )
    def sc_pool(x_hbm, table_hbm, out_hbm, idx_v, r0, r1, r2, r3,
                r4, r5, r6, r7, pool_v, s0, s1, s2, s3, s4, s5, s6, s7):
        rows = (r0, r1, r2, r3, r4, r5, r6, r7)
        sems = (s0, s1, s2, s3, s4, s5, s6, s7)
        chlen = (CH0, CH1)
        choff = (0, CH0)
        wid = lax.axis_index("s") * NC + lax.axis_index("c")
        base = wid * B_PER_W
        pltpu.sync_copy(x_hbm.at[pl.ds(base, B_PER_W)], idx_v)

        for b in range(RING):
            pltpu.async_copy(
                table_hbm.at[idx_v.at[b // 2, pl.ds(choff[b % 2],
                                                    chlen[b % 2])]],
                rows[b], sems[b])

        scale = jnp.float32(1.0 / SEQ)
        zero = jnp.zeros((16,), jnp.float32)
        samples_per_group = RING // 2

        def outer(t, _):
            for k in range(samples_per_group):
                s = samples_per_group * t + k
                accs = (zero, zero, zero, zero)
                for hb in range(2):
                    b = 2 * k + hb
                    pltpu.make_async_copy(
                        table_hbm.at[pl.ds(0, chlen[hb])],
                        rows[b], sems[b]).wait()

                    rbuf = rows[b]

                    def acc_body(r, a, rbuf=rbuf):
                        return (
                            a[0] + rbuf[r, pl.ds(0, 16)],
                            a[1] + rbuf[r, pl.ds(16, 16)],
                            a[2] + rbuf[r, pl.ds(32, 16)],
                            a[3] + rbuf[r, pl.ds(48, 16)],
                        )

                    accs = lax.fori_loop(0, chlen[hb], acc_body, accs,
                                         unroll=8)

                    s_next = s + samples_per_group

                    @pl.when(s_next < B_PER_W)
                    def _(b=b, hb=hb, s_next=s_next):
                        pltpu.async_copy(
                            table_hbm.at[idx_v.at[s_next,
                                                  pl.ds(choff[hb],
                                                        chlen[hb])]],
                            rows[b], sems[b])

                pool_v[s, pl.ds(0, 16)] = accs[0] * scale
                pool_v[s, pl.ds(16, 16)] = accs[1] * scale
                pool_v[s, pl.ds(32, 16)] = accs[2] * scale
                pool_v[s, pl.ds(48, 16)] = accs[3] * scale
            return 0

        lax.fori_loop(0, B_PER_W // samples_per_group, outer, 0)
        pltpu.sync_copy(pool_v, out_hbm.at[pl.ds(base, B_PER_W)])

    return sc_pool(x2, table)


def _mlp_body(m_ref, w1_ref, b1_ref, w2_ref, b2_ref, o_ref):
    h = jnp.dot(m_ref[...], w1_ref[...],
                preferred_element_type=jnp.float32) + b1_ref[...]
    h = jax.nn.sigmoid(jnp.maximum(h, 0.0))
    o = jnp.dot(h, w2_ref[...],
                preferred_element_type=jnp.float32) + b2_ref[...]
    o_ref[...] = jax.nn.sigmoid(o)


def _mlp_call(pooled, W1, b1, W2, b2):
    return pl.pallas_call(
        _mlp_body,
        out_shape=jax.ShapeDtypeStruct((B, 1), jnp.float32),
    )(pooled, W1, b1.reshape(1, 16), W2, b2.reshape(1, 1))


def kernel(x, table, W1, b1, W2, b2):
    pooled = _sc_pool_call(x.astype(jnp.int32), table)
    return _mlp_call(pooled, W1, b1, W2, b2)
